# baseline (speedup 1.0000x reference)
import functools

import jax
import jax.numpy as jnp
from jax import lax
from jax.experimental import pallas as pl
from jax.experimental.pallas import tpu as pltpu
from jax.experimental.pallas import tpu_sc as plsc

N = 10000
C = 32
D = 128

NUM_WORKERS = 32
N_PAD = 10240
NODES_PER_W = N_PAD // NUM_WORKERS
G = 8
GROUPS = NODES_PER_W // G
HALF_IDX = G * C // 2
IDX_ROWS = NODES_PER_W * C // HALF_IDX


def _sc_body(h_hbm, ci_hbm, s_hbm, r_hbm,
             idx_all, rows, sout0, rout0, sout1, rout1,
             g0a, g0b, g1a, g1b, ss0, rs0, ss1, rs1):
    wid = lax.axis_index("s") * 2 + lax.axis_index("c")
    wbase = wid * NODES_PER_W

    def compute_half(rows_ref, sout, rout, i0):
        def body(tj, _):
            i_loc = tj // 8
            j16 = pl.multiple_of((tj % 8) * 16, 16)
            rbase = i_loc * C
            s_acc = [jnp.zeros((16,), jnp.float32) for _ in range(4)]
            r_acc = [jnp.zeros((16,), jnp.float32) for _ in range(4)]
            for c in range(C):
                k = c & 3
                row = rows_ref[rbase + c, pl.ds(j16, 16)]
                s_acc[k] = s_acc[k] + row
                r_acc[k] = r_acc[k] + float(c) * row
            s = (s_acc[0] + s_acc[1]) + (s_acc[2] + s_acc[3])
            r = (r_acc[0] + r_acc[1]) + (r_acc[2] + r_acc[3])
            sout[i0 + i_loc, pl.ds(j16, 16)] = s
            rout[i0 + i_loc, pl.ds(j16, 16)] = r
            return _
        lax.fori_loop(0, (G // 2) * 8, body, 0)

    def compute(rows_a, rows_b, sout, rout):
        pass

    def start_gathers(g, ra, rb, sa, sb):
        a = pltpu.async_copy(h_hbm.at[idx_all.at[2 * g]], ra, sa)
        b = pltpu.async_copy(h_hbm.at[idx_all.at[2 * g + 1]], rb, sb)
        return a, b

    def wait_gathers(g, ra, rb, sa, sb):
        pltpu.make_async_copy(h_hbm.at[idx_all.at[2 * g]], ra, sa).wait()
        pltpu.make_async_copy(h_hbm.at[idx_all.at[2 * g + 1]], rb, sb).wait()

    def store(g, sout, rout, ssem, rsem):
        base = pl.multiple_of(wbase + g * G, 8)
        a = pltpu.async_copy(sout, s_hbm.at[pl.ds(base, G)], ssem)
        b = pltpu.async_copy(rout, r_hbm.at[pl.ds(base, G)], rsem)
        return a, b

    def wait_store(g, sout, rout, ssem, rsem):
        base = pl.multiple_of(wbase + g * G, 8)
        pltpu.make_async_copy(sout, s_hbm.at[pl.ds(base, G)], ssem).wait()
        pltpu.make_async_copy(rout, r_hbm.at[pl.ds(base, G)], rsem).wait()

    pltpu.sync_copy(ci_hbm.at[pl.ds(wid * IDX_ROWS, IDX_ROWS)], idx_all)

    NBUF = 5
    sems = [g0a, g0b, g1a, g1b, ss0]
    for b in range(NBUF):
        pltpu.async_copy(h_hbm.at[idx_all.at[b]], rows.at[b], sems[b])

    def ring_body(it, _):
        for b in range(NBUF):
            sidx = it * NBUF + b
            pltpu.make_async_copy(
                h_hbm.at[idx_all.at[sidx]], rows.at[b], sems[b]).wait()
            nxt = jnp.minimum(sidx + NBUF, IDX_ROWS - 1)
            pltpu.async_copy(h_hbm.at[idx_all.at[nxt]], rows.at[b], sems[b])
        return _

    lax.fori_loop(0, IDX_ROWS // NBUF, ring_body, 0)
    for b in range(NBUF):
        pltpu.make_async_copy(
            h_hbm.at[idx_all.at[IDX_ROWS - 1]], rows.at[b], sems[b]).wait()
    return

    start_gathers(0, rows.at[0], rows.at[1], g0a, g0b)

    def it_body(it, _):
        geven = 2 * it
        godd = geven + 1
        gnext = jnp.minimum(geven + 2, GROUPS - 1)
        start_gathers(godd, rows.at[2], rows.at[3], g1a, g1b)
        wait_gathers(geven, rows.at[0], rows.at[1], g0a, g0b)

        @pl.when(it > 0)
        def _w0():
            wait_store(geven - 2, sout0, rout0, ss0, rs0)
        compute(rows.at[0], rows.at[1], sout0, rout0)
        store(geven, sout0, rout0, ss0, rs0)

        start_gathers(gnext, rows.at[0], rows.at[1], g0a, g0b)
        wait_gathers(godd, rows.at[2], rows.at[3], g1a, g1b)

        @pl.when(it > 0)
        def _w1():
            wait_store(godd - 2, sout1, rout1, ss1, rs1)
        compute(rows.at[2], rows.at[3], sout1, rout1)
        store(godd, sout1, rout1, ss1, rs1)
        return _

    lax.fori_loop(0, GROUPS // 2, it_body, 0)

    wait_gathers(GROUPS - 1, rows.at[0], rows.at[1], g0a, g0b)
    wait_store(GROUPS - 2, sout0, rout0, ss0, rs0)
    wait_store(GROUPS - 1, sout1, rout1, ss1, rs1)


@functools.cache
def _make_sc_call():
    return functools.partial(
        pl.kernel,
        out_type=(
            jax.ShapeDtypeStruct((N_PAD, D), jnp.float32),
            jax.ShapeDtypeStruct((N_PAD, D), jnp.float32),
        ),
        mesh=plsc.VectorSubcoreMesh(core_axis_name="c", subcore_axis_name="s"),
        scratch_types=[
            pltpu.VMEM((IDX_ROWS, HALF_IDX), jnp.int32),
            pltpu.VMEM((5, HALF_IDX, D), jnp.float32),
            pltpu.VMEM((G, D), jnp.float32),
            pltpu.VMEM((G, D), jnp.float32),
            pltpu.VMEM((G, D), jnp.float32),
            pltpu.VMEM((G, D), jnp.float32),
            pltpu.SemaphoreType.DMA,
            pltpu.SemaphoreType.DMA,
            pltpu.SemaphoreType.DMA,
            pltpu.SemaphoreType.DMA,
            pltpu.SemaphoreType.DMA,
            pltpu.SemaphoreType.DMA,
            pltpu.SemaphoreType.DMA,
            pltpu.SemaphoreType.DMA,
        ],
    )(_sc_body)


def _tc_body(s_ref, r_ref, h_ref, wl_ref, wr_ref, wt_ref, b_ref, o_ref):
    wd = (wr_ref[...] - wl_ref[...]) * (1.0 / (C - 1))
    acc = jnp.dot(s_ref[...], wl_ref[...], preferred_element_type=jnp.float32)
    acc = acc + jnp.dot(r_ref[...], wd, preferred_element_type=jnp.float32)
    acc = acc + jnp.dot(h_ref[...], wt_ref[...], preferred_element_type=jnp.float32)
    o_ref[...] = jnp.maximum(acc + b_ref[...], 0.0)


TC_BLOCK = 1024


def _tc_call(s, r, h_pad, wl, wr, wt, b):
    grid = (N_PAD // TC_BLOCK,)
    row_spec = pl.BlockSpec((TC_BLOCK, D), lambda i: (i, 0))
    w_spec = pl.BlockSpec((D, D), lambda i: (0, 0))
    return pl.pallas_call(
        _tc_body,
        grid=grid,
        in_specs=[row_spec, row_spec, row_spec, w_spec, w_spec, w_spec,
                  pl.BlockSpec((1, D), lambda i: (0, 0))],
        out_specs=row_spec,
        out_shape=jax.ShapeDtypeStruct((N_PAD, D), jnp.float32),
    )(s, r, h_pad, wl, wr, wt, b)


def kernel(h, child_idx, W_left, W_right, W_top, b_conv):
    ci = child_idx.astype(jnp.int32)
    ci = jnp.pad(ci, ((0, N_PAD - N), (0, 0)))
    ci_2d = ci.reshape(N_PAD * C // HALF_IDX, HALF_IDX)
    s, r = _make_sc_call()(h, ci_2d)
    h_pad = jnp.pad(h, ((0, N_PAD - N), (0, 0)))
    out = _tc_call(s, r, h_pad, W_left, W_right, W_top, b_conv)
    return out[:N]


# speedup vs baseline: 3.3358x; 3.3358x over previous
import functools

import jax
import jax.numpy as jnp
from jax import lax
from jax.experimental import pallas as pl
from jax.experimental.pallas import tpu as pltpu
from jax.experimental.pallas import tpu_sc as plsc

N = 10000
C = 32
D = 128

NUM_WORKERS = 32
N_PAD = 10240
NODES_PER_W = N_PAD // NUM_WORKERS
G = 8
GROUPS_2 = NODES_PER_W // G
HALF_IDX = G * C // 2
IDX_ROWS = NODES_PER_W * C // HALF_IDX
PAIR_ROWS = N_PAD // 2
FILL_CHUNK = 32
FILL_PER_TILE = PAIR_ROWS // 16


def _sc_body(hp_hbm, ci_hbm, par_hbm, s_hbm, r_hbm,
             h_sh, idx_all, par_all, rows, fb, sout, rout,
             g0a, g0b, ss0, rs0):
    sid = lax.axis_index("s")
    wid = sid * 2 + lax.axis_index("c")
    wbase = wid * NODES_PER_W

    fbase = sid * FILL_PER_TILE
    for k in range(FILL_PER_TILE // FILL_CHUNK):
        off = pl.multiple_of(fbase + k * FILL_CHUNK, 8)
        pltpu.sync_copy(hp_hbm.at[pl.ds(off, FILL_CHUNK)], fb)
        pltpu.sync_copy(fb, h_sh.at[pl.ds(off, FILL_CHUNK)])
    pltpu.sync_copy(ci_hbm.at[pl.ds(wid * IDX_ROWS, IDX_ROWS)], idx_all)
    pltpu.sync_copy(par_hbm.at[pl.ds(wid * IDX_ROWS, IDX_ROWS)], par_all)
    plsc.subcore_barrier()

    def compute_half(rows_ref, prow, sout, rout, i0):
        def body(i_loc, _):
            rbase = i_loc * C
            zero = jnp.zeros((16,), jnp.float32)
            t = [zero] * 8
            r = [zero] * 8
            pv_lo = par_all[prow, pl.ds(pl.multiple_of(rbase, 32), 16)]
            pv_hi = par_all[prow, pl.ds(pl.multiple_of(rbase + 16, 16), 16)]
            for c in range(C - 1, -1, -1):
                par = (pv_hi if c >= 16 else pv_lo)[c & 15]
                sh = 16 - (par << 4)
                for w in range(8):
                    wd = rows_ref[rbase + c, pl.ds(w * 16, 16)]
                    wi = lax.bitcast_convert_type(wd, jnp.int32)
                    val = lax.bitcast_convert_type(wi << sh, jnp.float32)
                    t[w] = t[w] + val
                    r[w] = r[w] + t[w]
            for w in range(8):
                w16 = 16 * w
                sout[i0 + i_loc, pl.ds(w16, 16)] = t[w]
                rout[i0 + i_loc, pl.ds(w16, 16)] = r[w] - t[w]
            return _
        lax.fori_loop(0, G // 2, body, 0)

    def start_stream(s, buf, sem):
        return pltpu.async_copy(h_sh.at[idx_all.at[s]], buf, sem)

    def wait_stream(s, buf, sem):
        pltpu.make_async_copy(h_sh.at[idx_all.at[s]], buf, sem).wait()

    def store(it):
        base = pl.multiple_of(wbase + it * G, 8)
        pltpu.async_copy(sout, s_hbm.at[pl.ds(base, G)], ss0)
        pltpu.async_copy(rout, r_hbm.at[pl.ds(base, G)], rs0)

    def wait_store(it):
        base = pl.multiple_of(wbase + it * G, 8)
        pltpu.make_async_copy(sout, s_hbm.at[pl.ds(base, G)], ss0).wait()
        pltpu.make_async_copy(rout, r_hbm.at[pl.ds(base, G)], rs0).wait()

    NSTREAM = 2 * GROUPS_2
    start_stream(0, rows.at[0], g0a)
    start_stream(1, rows.at[1], g0b)

    def it_body(it, _):
        s0 = 2 * it
        s1 = s0 + 1
        wait_stream(s0, rows.at[0], g0a)

        @pl.when(it > 0)
        def _ws():
            wait_store(it - 1)
        compute_half(rows.at[0], s0, sout, rout, 0)
        start_stream(jnp.minimum(s0 + 2, NSTREAM - 1), rows.at[0], g0a)
        wait_stream(s1, rows.at[1], g0b)
        compute_half(rows.at[1], s1, sout, rout, G // 2)
        start_stream(jnp.minimum(s1 + 2, NSTREAM - 1), rows.at[1], g0b)
        store(it)
        return _

    lax.fori_loop(0, GROUPS_2, it_body, 0)

    wait_stream(NSTREAM - 1, rows.at[0], g0a)
    wait_stream(NSTREAM - 1, rows.at[1], g0b)
    wait_store(GROUPS_2 - 1)

@functools.cache
def _make_sc_call():
    return functools.partial(
        pl.kernel,
        out_type=(
            jax.ShapeDtypeStruct((N_PAD, D), jnp.float32),
            jax.ShapeDtypeStruct((N_PAD, D), jnp.float32),
        ),
        mesh=plsc.VectorSubcoreMesh(core_axis_name="c", subcore_axis_name="s"),
        scratch_types=[
            pltpu.VMEM_SHARED((PAIR_ROWS, D), jnp.float32),
            pltpu.VMEM((IDX_ROWS, HALF_IDX), jnp.int32),
            pltpu.VMEM((IDX_ROWS, HALF_IDX), jnp.int32),
            pltpu.VMEM((2, HALF_IDX, D), jnp.float32),
            pltpu.VMEM((FILL_CHUNK, D), jnp.float32),
            pltpu.VMEM((G, D), jnp.float32),
            pltpu.VMEM((G, D), jnp.float32),
            pltpu.SemaphoreType.DMA,
            pltpu.SemaphoreType.DMA,
            pltpu.SemaphoreType.DMA,
            pltpu.SemaphoreType.DMA,
        ],
    )(_sc_body)


def _tc_body(s_ref, r_ref, h_ref, wl_ref, wr_ref, wt_ref, b_ref, o_ref):
    wd = (wr_ref[...] - wl_ref[...]) * (1.0 / (C - 1))
    acc = jnp.dot(s_ref[...], wl_ref[...], preferred_element_type=jnp.float32)
    acc = acc + jnp.dot(r_ref[...], wd, preferred_element_type=jnp.float32)
    acc = acc + jnp.dot(h_ref[...], wt_ref[...], preferred_element_type=jnp.float32)
    o_ref[...] = jnp.maximum(acc + b_ref[...], 0.0)


TC_BLOCK = 1024


def _tc_call(s, r, h_pad, wl, wr, wt, b):
    grid = (N_PAD // TC_BLOCK,)
    row_spec = pl.BlockSpec((TC_BLOCK, D), lambda i: (i, 0))
    w_spec = pl.BlockSpec((D, D), lambda i: (0, 0))
    return pl.pallas_call(
        _tc_body,
        grid=grid,
        in_specs=[row_spec, row_spec, row_spec, w_spec, w_spec, w_spec,
                  pl.BlockSpec((1, D), lambda i: (0, 0))],
        out_specs=row_spec,
        out_shape=jax.ShapeDtypeStruct((N_PAD, D), jnp.float32),
    )(s, r, h_pad, wl, wr, wt, b)


def kernel(h, child_idx, W_left, W_right, W_top, b_conv):
    ci = child_idx.astype(jnp.int32)
    ci = jnp.pad(ci, ((0, N_PAD - N), (0, 0)))
    ci_row = (ci >> 1).reshape(N_PAD * C // HALF_IDX, HALF_IDX)
    ci_par = (ci & 1).reshape(N_PAD * C // HALF_IDX, HALF_IDX)
    h_pad = jnp.pad(h, ((0, N_PAD - N), (0, 0)))
    h_bf = h_pad.astype(jnp.bfloat16).reshape(PAIR_ROWS, 2, D)
    hp32 = lax.bitcast_convert_type(h_bf.transpose(0, 2, 1), jnp.float32)
    s, r = _make_sc_call()(hp32, ci_row, ci_par)
    out = _tc_call(s, r, h_pad, W_left, W_right, W_top, b_conv)
    return out[:N]
